# baseline (speedup 1.0000x reference)
import functools

import jax
import jax.numpy as jnp
import numpy as np
from jax.experimental import pallas as pl
from jax.experimental.pallas import tpu as pltpu


def _neighbor_body(b_ref, a1_ref, a2_ref, a3_ref, p_ref, items_bf_ref,
                   i1_ref, i2_ref, i3_ref, *unused_refs_out, inv_sqrt_d):
    unused_refs = unused_refs_out[:10]
    out_ref, gate_ref = unused_refs_out[10], unused_refs_out[11]
    m = pl.program_id(0)

    @pl.when(m == 0)
    def _():
        z = jnp.float32(0.0)
        for r in unused_refs:
            z += jnp.sum(r[...])
        g = jnp.where(b_ref[0] == 2, 1.0 - z, 0.0)
        gate_ref[0, 0] = g

    rhs = items_bf_ref[...]
    n1 = jnp.dot(a1_ref[...].astype(jnp.bfloat16), rhs,
                 preferred_element_type=jnp.float32)
    n2 = jnp.dot(a2_ref[...].astype(jnp.bfloat16), rhs,
                 preferred_element_type=jnp.float32)
    n3 = jnp.dot(a3_ref[...].astype(jnp.bfloat16), rhs,
                 preferred_element_type=jnp.float32)

    p = p_ref[...]

    def score(n, i_ref):
        a = jnp.dot(p * n, i_ref[...], preferred_element_type=jnp.float32)
        a = jnp.where(a >= 0, a, 0.2 * a)
        return jnp.sum(a, axis=1, keepdims=True) * inv_sqrt_d

    s1 = score(n1, i1_ref)
    s2 = score(n2, i2_ref)
    s3 = score(n3, i3_ref)
    mx = jnp.maximum(jnp.maximum(s1, s2), s3)
    e1 = jnp.exp(s1 - mx)
    e2 = jnp.exp(s2 - mx)
    e3 = jnp.exp(s3 - mx)
    scale = gate_ref[0, 0] / (e1 + e2 + e3)
    out_ref[...] = (n1 * e1 + n2 * e2 + n3 * e3) * scale


def _user_body(gate_ref, ui_t_ref, items_bf_ref, out_ref, *, k_blk):
    k = pl.program_id(0)
    nk = pl.num_programs(0)

    @pl.when(k == 0)
    def _():
        out_ref[...] = jnp.zeros_like(out_ref)

    rhs = items_bf_ref[pl.ds(k * k_blk, k_blk), :]
    out_ref[...] += jax.lax.dot_general(
        ui_t_ref[...].astype(jnp.bfloat16), rhs,
        (((0,), (0,)), ((), ())), preferred_element_type=jnp.float32)

    @pl.when(k == nk - 1)
    def _():
        out_ref[...] *= gate_ref[0, 0]


def kernel(b, users_emb, items_emb, e2e_in, e2e_out, p2p_in, p2p_out, e2p_in, e2p_out, iu, iu_p, iu_c, ui, ui_p, ui_c, uu_p, uu_c, I_p2p_in, I_p2p_out, I_e2p_in, I_e2e_in, I_e2e_out, I_e2p_out):
    n_items, d = items_emb.shape
    n_users = ui_p.shape[0]
    inv_sqrt_d = float(1.0 / np.sqrt(d))

    b_arr = jnp.asarray(b, jnp.int32).reshape((1,))
    items_bf = items_emb.astype(jnp.bfloat16)
    ui_t = ui_p.T

    m_blk = 200
    n_m = n_items // m_blk

    unused = [e2e_in, e2e_out, e2p_out, iu, iu_p, iu_c, ui, ui_c, uu_p, uu_c]

    neighbor, gate = pl.pallas_call(
        functools.partial(_neighbor_body, inv_sqrt_d=inv_sqrt_d),
        grid=(n_m,),
        in_specs=[
            pl.BlockSpec(memory_space=pltpu.SMEM),
            pl.BlockSpec((m_blk, n_items), lambda m: (m, 0)),
            pl.BlockSpec((m_blk, n_items), lambda m: (m, 0)),
            pl.BlockSpec((m_blk, n_items), lambda m: (m, 0)),
            pl.BlockSpec((m_blk, d), lambda m: (m, 0)),
            pl.BlockSpec(memory_space=pltpu.VMEM),
            pl.BlockSpec(memory_space=pltpu.VMEM),
            pl.BlockSpec(memory_space=pltpu.VMEM),
            pl.BlockSpec(memory_space=pltpu.VMEM),
        ] + [pl.BlockSpec(memory_space=pltpu.VMEM)] * 10,
        out_specs=[
            pl.BlockSpec((m_blk, d), lambda m: (m, 0)),
            pl.BlockSpec(memory_space=pltpu.SMEM),
        ],
        out_shape=[
            jax.ShapeDtypeStruct((n_items, d), jnp.float32),
            jax.ShapeDtypeStruct((1, 1), jnp.float32),
        ],
        compiler_params=pltpu.CompilerParams(
            dimension_semantics=("arbitrary",),
        ),
    )(b_arr, p2p_in, p2p_out, e2p_in, items_emb, items_bf,
      I_p2p_in, I_p2p_out, I_e2p_in, *unused)

    k_blk = 400
    n_k = n_items // k_blk

    u_emb_ui = pl.pallas_call(
        functools.partial(_user_body, k_blk=k_blk),
        grid=(n_k,),
        in_specs=[
            pl.BlockSpec(memory_space=pltpu.SMEM),
            pl.BlockSpec((k_blk, n_users), lambda k: (k, 0)),
            pl.BlockSpec(memory_space=pltpu.VMEM),
        ],
        out_specs=pl.BlockSpec((n_users, d), lambda k: (0, 0)),
        out_shape=jax.ShapeDtypeStruct((n_users, d), jnp.float32),
        compiler_params=pltpu.CompilerParams(
            dimension_semantics=("arbitrary",),
        ),
    )(gate, ui_t, items_bf)

    return (u_emb_ui, neighbor)


# speedup vs baseline: 1.0123x; 1.0123x over previous
import functools

import jax
import jax.numpy as jnp
import numpy as np
from jax.experimental import pallas as pl
from jax.experimental.pallas import tpu as pltpu


def _neighbor_body(b_ref, a1_ref, a2_ref, a3_ref, p_ref, items_bf_ref,
                   i1_ref, i2_ref, i3_ref, *unused_refs_out, inv_sqrt_d):
    unused_refs = unused_refs_out[:10]
    out_ref, gate_ref = unused_refs_out[10], unused_refs_out[11]
    m = pl.program_id(0)

    @pl.when(m == 0)
    def _():
        z = jnp.float32(0.0)
        for r in unused_refs:
            z += jnp.sum(r[...])
        g = jnp.where(b_ref[0] == 2, 1.0 - z, 0.0)
        gate_ref[0, 0] = g

    rhs = items_bf_ref[...]
    n1 = jnp.dot(a1_ref[...].astype(jnp.bfloat16), rhs,
                 preferred_element_type=jnp.float32)
    n2 = jnp.dot(a2_ref[...].astype(jnp.bfloat16), rhs,
                 preferred_element_type=jnp.float32)
    n3 = jnp.dot(a3_ref[...].astype(jnp.bfloat16), rhs,
                 preferred_element_type=jnp.float32)

    p = p_ref[...]

    def score(n, i_ref):
        a = jnp.dot(p * n, i_ref[...], preferred_element_type=jnp.float32)
        a = jnp.where(a >= 0, a, 0.2 * a)
        return jnp.sum(a, axis=1, keepdims=True) * inv_sqrt_d

    s1 = score(n1, i1_ref)
    s2 = score(n2, i2_ref)
    s3 = score(n3, i3_ref)
    mx = jnp.maximum(jnp.maximum(s1, s2), s3)
    e1 = jnp.exp(s1 - mx)
    e2 = jnp.exp(s2 - mx)
    e3 = jnp.exp(s3 - mx)
    scale = gate_ref[0, 0] / (e1 + e2 + e3)
    out_ref[...] = (n1 * e1 + n2 * e2 + n3 * e3) * scale


def _user_body(gate_ref, ui_t_ref, items_bf_ref, out_ref, acc_ref, *, k_blk):
    k = pl.program_id(0)
    nk = pl.num_programs(0)

    @pl.when(k == 0)
    def _():
        acc_ref[...] = jnp.zeros_like(acc_ref)

    lhs = items_bf_ref[pl.ds(k * k_blk, k_blk), :]
    acc_ref[...] += jax.lax.dot_general(
        lhs, ui_t_ref[...].astype(jnp.bfloat16),
        (((0,), (0,)), ((), ())), preferred_element_type=jnp.float32)

    @pl.when(k == nk - 1)
    def _():
        out_ref[...] = acc_ref[...].T * gate_ref[0, 0]


def kernel(b, users_emb, items_emb, e2e_in, e2e_out, p2p_in, p2p_out, e2p_in, e2p_out, iu, iu_p, iu_c, ui, ui_p, ui_c, uu_p, uu_c, I_p2p_in, I_p2p_out, I_e2p_in, I_e2e_in, I_e2e_out, I_e2p_out):
    n_items, d = items_emb.shape
    n_users = ui_p.shape[0]
    inv_sqrt_d = float(1.0 / np.sqrt(d))

    b_arr = jnp.asarray(b, jnp.int32).reshape((1,))
    items_bf = items_emb.astype(jnp.bfloat16)
    ui_t = ui_p.T

    m_blk = 200
    n_m = n_items // m_blk

    unused = [e2e_in, e2e_out, e2p_out, iu, iu_p, iu_c, ui, ui_c, uu_p, uu_c]

    neighbor, gate = pl.pallas_call(
        functools.partial(_neighbor_body, inv_sqrt_d=inv_sqrt_d),
        grid=(n_m,),
        in_specs=[
            pl.BlockSpec(memory_space=pltpu.SMEM),
            pl.BlockSpec((m_blk, n_items), lambda m: (m, 0)),
            pl.BlockSpec((m_blk, n_items), lambda m: (m, 0)),
            pl.BlockSpec((m_blk, n_items), lambda m: (m, 0)),
            pl.BlockSpec((m_blk, d), lambda m: (m, 0)),
            pl.BlockSpec(memory_space=pltpu.VMEM),
            pl.BlockSpec(memory_space=pltpu.VMEM),
            pl.BlockSpec(memory_space=pltpu.VMEM),
            pl.BlockSpec(memory_space=pltpu.VMEM),
        ] + [pl.BlockSpec(memory_space=pltpu.VMEM)] * 10,
        out_specs=[
            pl.BlockSpec((m_blk, d), lambda m: (m, 0)),
            pl.BlockSpec(memory_space=pltpu.SMEM),
        ],
        out_shape=[
            jax.ShapeDtypeStruct((n_items, d), jnp.float32),
            jax.ShapeDtypeStruct((1, 1), jnp.float32),
        ],
        compiler_params=pltpu.CompilerParams(
            dimension_semantics=("arbitrary",),
        ),
    )(b_arr, p2p_in, p2p_out, e2p_in, items_emb, items_bf,
      I_p2p_in, I_p2p_out, I_e2p_in, *unused)

    k_blk = 400
    n_k = n_items // k_blk

    u_emb_ui = pl.pallas_call(
        functools.partial(_user_body, k_blk=k_blk),
        grid=(n_k,),
        in_specs=[
            pl.BlockSpec(memory_space=pltpu.SMEM),
            pl.BlockSpec((k_blk, n_users), lambda k: (k, 0)),
            pl.BlockSpec(memory_space=pltpu.VMEM),
        ],
        out_specs=pl.BlockSpec((n_users, d), lambda k: (0, 0)),
        out_shape=jax.ShapeDtypeStruct((n_users, d), jnp.float32),
        scratch_shapes=[pltpu.VMEM((d, n_users), jnp.float32)],
        compiler_params=pltpu.CompilerParams(
            dimension_semantics=("arbitrary",),
        ),
    )(gate, ui_t, items_bf)

    return (u_emb_ui, neighbor)


# speedup vs baseline: 1.0214x; 1.0089x over previous
import functools

import jax
import jax.numpy as jnp
import numpy as np
from jax.experimental import pallas as pl
from jax.experimental.pallas import tpu as pltpu


def _neighbor_body(b_ref, a1_ref, a2_ref, a3_ref, items_ref,
                   i1_ref, i2_ref, i3_ref, *unused_refs_out,
                   inv_sqrt_d, m_blk):
    unused_refs = unused_refs_out[:10]
    out_ref, gate_ref, items_bf_ref = unused_refs_out[10:13]
    m = pl.program_id(0)

    @pl.when(m == 0)
    def _():
        z = jnp.float32(0.0)
        for r in unused_refs:
            z += jnp.sum(r[...])
        g = jnp.where(b_ref[0] == 2, 1.0 - z, 0.0)
        gate_ref[0, 0] = g
        items_bf_ref[...] = items_ref[...].astype(jnp.bfloat16)

    rhs = items_bf_ref[...]
    n1 = jnp.dot(a1_ref[...].astype(jnp.bfloat16), rhs,
                 preferred_element_type=jnp.float32)
    n2 = jnp.dot(a2_ref[...].astype(jnp.bfloat16), rhs,
                 preferred_element_type=jnp.float32)
    n3 = jnp.dot(a3_ref[...].astype(jnp.bfloat16), rhs,
                 preferred_element_type=jnp.float32)

    p = items_ref[pl.ds(m * m_blk, m_blk), :]

    def score(n, i_ref):
        a = jnp.dot(p * n, i_ref[...], preferred_element_type=jnp.float32)
        a = jnp.where(a >= 0, a, 0.2 * a)
        return jnp.sum(a, axis=1, keepdims=True) * inv_sqrt_d

    s1 = score(n1, i1_ref)
    s2 = score(n2, i2_ref)
    s3 = score(n3, i3_ref)
    mx = jnp.maximum(jnp.maximum(s1, s2), s3)
    e1 = jnp.exp(s1 - mx)
    e2 = jnp.exp(s2 - mx)
    e3 = jnp.exp(s3 - mx)
    scale = gate_ref[0, 0] / (e1 + e2 + e3)
    out_ref[...] = (n1 * e1 + n2 * e2 + n3 * e3) * scale


def _user_body(gate_ref, ui_t_ref, items_bf_ref, out_ref, acc_ref, *, k_blk):
    k = pl.program_id(0)
    nk = pl.num_programs(0)

    @pl.when(k == 0)
    def _():
        acc_ref[...] = jnp.zeros_like(acc_ref)

    lhs = items_bf_ref[pl.ds(k * k_blk, k_blk), :]
    acc_ref[...] += jax.lax.dot_general(
        lhs, ui_t_ref[...].astype(jnp.bfloat16),
        (((0,), (0,)), ((), ())), preferred_element_type=jnp.float32)

    @pl.when(k == nk - 1)
    def _():
        out_ref[...] = acc_ref[...].T * gate_ref[0, 0]


def kernel(b, users_emb, items_emb, e2e_in, e2e_out, p2p_in, p2p_out, e2p_in, e2p_out, iu, iu_p, iu_c, ui, ui_p, ui_c, uu_p, uu_c, I_p2p_in, I_p2p_out, I_e2p_in, I_e2e_in, I_e2e_out, I_e2p_out):
    n_items, d = items_emb.shape
    n_users = ui_p.shape[0]
    inv_sqrt_d = float(1.0 / np.sqrt(d))

    b_arr = jnp.asarray(b, jnp.int32).reshape((1,))
    ui_t = ui_p.T

    m_blk = 200
    n_m = n_items // m_blk

    unused = [e2e_in, e2e_out, e2p_out, iu, iu_p, iu_c, ui, ui_c, uu_p, uu_c]

    neighbor, gate, items_bf = pl.pallas_call(
        functools.partial(_neighbor_body, inv_sqrt_d=inv_sqrt_d, m_blk=m_blk),
        grid=(n_m,),
        in_specs=[
            pl.BlockSpec(memory_space=pltpu.SMEM),
            pl.BlockSpec((m_blk, n_items), lambda m: (m, 0)),
            pl.BlockSpec((m_blk, n_items), lambda m: (m, 0)),
            pl.BlockSpec((m_blk, n_items), lambda m: (m, 0)),
            pl.BlockSpec(memory_space=pltpu.VMEM),
            pl.BlockSpec(memory_space=pltpu.VMEM),
            pl.BlockSpec(memory_space=pltpu.VMEM),
            pl.BlockSpec(memory_space=pltpu.VMEM),
        ] + [pl.BlockSpec(memory_space=pltpu.VMEM)] * 10,
        out_specs=[
            pl.BlockSpec((m_blk, d), lambda m: (m, 0)),
            pl.BlockSpec(memory_space=pltpu.SMEM),
            pl.BlockSpec(memory_space=pltpu.VMEM),
        ],
        out_shape=[
            jax.ShapeDtypeStruct((n_items, d), jnp.float32),
            jax.ShapeDtypeStruct((1, 1), jnp.float32),
            jax.ShapeDtypeStruct((n_items, d), jnp.bfloat16),
        ],
        compiler_params=pltpu.CompilerParams(
            dimension_semantics=("arbitrary",),
        ),
    )(b_arr, p2p_in, p2p_out, e2p_in, items_emb,
      I_p2p_in, I_p2p_out, I_e2p_in, *unused)

    k_blk = 400
    n_k = n_items // k_blk

    u_emb_ui = pl.pallas_call(
        functools.partial(_user_body, k_blk=k_blk),
        grid=(n_k,),
        in_specs=[
            pl.BlockSpec(memory_space=pltpu.SMEM),
            pl.BlockSpec((k_blk, n_users), lambda k: (k, 0)),
            pl.BlockSpec(memory_space=pltpu.VMEM),
        ],
        out_specs=pl.BlockSpec((n_users, d), lambda k: (0, 0)),
        out_shape=jax.ShapeDtypeStruct((n_users, d), jnp.float32),
        scratch_shapes=[pltpu.VMEM((d, n_users), jnp.float32)],
        compiler_params=pltpu.CompilerParams(
            dimension_semantics=("arbitrary",),
        ),
    )(gate, ui_t, items_bf)

    return (u_emb_ui, neighbor)
